# baseline (speedup 1.0000x reference)
import jax
import jax.numpy as jnp
from jax import lax
from jax.experimental import pallas as pl
from jax.experimental.pallas import tpu as pltpu
from jax.experimental.pallas import tpu_sc as plsc

N = 10000
E = 160000
D = 256
DE = 16
H = 256
NC = 1024

NWORK = 32
EPW = E // NWORK
GCH = 72
SCH = 80
RSUB = 624
RTAIL = N - 16 * RSUB
HHALF = H // 2

EH1 = 83200
EH2 = E - EH1


def _node_prep_body(x_ref, wn1_ref, bn1_ref, wn2_ref, bn2_ref, wma_ref, wmb_ref,
                    h_ref, a_ref, b_ref):
    h = jax.nn.relu(jnp.dot(x_ref[...], wn1_ref[...],
                            preferred_element_type=jnp.float32) + bn1_ref[...])
    h = jax.nn.relu(jnp.dot(h, wn2_ref[...],
                            preferred_element_type=jnp.float32) + bn2_ref[...])
    h_ref[...] = h
    a_ref[...] = jnp.dot(h, wma_ref[...], preferred_element_type=jnp.float32)
    b_ref[...] = jnp.dot(h, wmb_ref[...], preferred_element_type=jnp.float32)


def _make_gather_body(epw, ngf, gtail):
    assert ngf % 3 == 0

    def _gather_body(a_hbm, b_hbm, src_hbm, dst_hbm, asrc_hbm, bdst_hbm,
                     sidx_v, didx_v, bufa0, bufb0, bufa1, bufb1, bufa2, bufb2,
                     gsa0, gsb0, gsa1, gsb1, gsa2, gsb2,
                     wsa0, wsb0, wsa1, wsb1, wsa2, wsb2):
        wid = lax.axis_index("s") * 2 + lax.axis_index("c")
        base = wid * epw
        pltpu.sync_copy(src_hbm.at[pl.ds(base, epw)], sidx_v)
        pltpu.sync_copy(dst_hbm.at[pl.ds(base, epw)], didx_v)

        bufa = (bufa0, bufa1, bufa2)
        bufb = (bufb0, bufb1, bufb2)
        gsa = (gsa0, gsa1, gsa2)
        gsb = (gsb0, gsb1, gsb2)
        wsa = (wsa0, wsa1, wsa2)
        wsb = (wsb0, wsb1, wsb2)

        def g_desc(off, slot):
            return (pltpu.make_async_copy(a_hbm.at[sidx_v.at[pl.ds(off, GCH)]],
                                          bufa[slot], gsa[slot]),
                    pltpu.make_async_copy(b_hbm.at[didx_v.at[pl.ds(off, GCH)]],
                                          bufb[slot], gsb[slot]))

        def w_desc(off, slot):
            return (pltpu.make_async_copy(
                        bufa[slot], asrc_hbm.at[pl.ds(base + off, GCH)], wsa[slot]),
                    pltpu.make_async_copy(
                        bufb[slot], bdst_hbm.at[pl.ds(base + off, GCH)], wsb[slot]))

        for d in g_desc(0, 0):
            d.start()
        for d in g_desc(GCH, 1):
            d.start()

        def phase(g, slot):
            off = g * GCH
            nslot = (slot + 2) % 3

            @pl.when(g + 2 < ngf)
            def _():
                @pl.when(g >= 1)
                def _():
                    for d in w_desc(off - GCH, nslot):
                        d.wait()
                for d in g_desc(off + 2 * GCH, nslot):
                    d.start()

            for d in g_desc(off, slot):
                d.wait()
            for d in w_desc(off, slot):
                d.start()

        @pl.loop(0, ngf // 3)
        def _(i):
            phase(3 * i, 0)
            phase(3 * i + 1, 1)
            phase(3 * i + 2, 2)

        for k in (ngf - 3, ngf - 2, ngf - 1):
            for d in w_desc(k * GCH, k % 3):
                d.wait()

        if gtail:
            toff = ngf * GCH
            pltpu.sync_copy(a_hbm.at[sidx_v.at[pl.ds(toff, gtail)]],
                            bufa0.at[pl.ds(0, gtail)])
            pltpu.sync_copy(b_hbm.at[didx_v.at[pl.ds(toff, gtail)]],
                            bufb0.at[pl.ds(0, gtail)])
            pltpu.sync_copy(bufa0.at[pl.ds(0, gtail)],
                            asrc_hbm.at[pl.ds(base + toff, gtail)])
            pltpu.sync_copy(bufb0.at[pl.ds(0, gtail)],
                            bdst_hbm.at[pl.ds(base + toff, gtail)])

    return _gather_body


def _edge_body(ea_ref, asrc_ref, bdst_ref, we_ref, be_ref, wmc_ref, bm_ref,
               wsp_ref, bsp_ref, m_ref, s_ref, p_ref):
    e = jax.nn.relu(jnp.dot(ea_ref[...], we_ref[...],
                            preferred_element_type=jnp.float32) + be_ref[...])
    ec = jnp.dot(e, wmc_ref[...], preferred_element_type=jnp.float32)
    m = jax.nn.relu(asrc_ref[...] + bdst_ref[...] + ec + bm_ref[...])
    m_ref[0, :, :] = m[:, :HHALF]
    m_ref[1, :, :] = m[:, HHALF:]
    u = e + m
    sp = jnp.dot(u, wsp_ref[...], preferred_element_type=jnp.float32) + bsp_ref[...]
    s_ref[0, 0, :] = jax.nn.sigmoid(sp[:, 0])
    p_ref[0, 0, :] = sp[:, 1]


def _make_scatter_body(eps, nsc):
    def _scatter_body(m_hbm, dstr_hbm, init_hbm, agg_hbm, idx_v, buf_v, buf2_v,
                      agg_sh, ls0, ls1, ss0, ss1):
        cid = lax.axis_index("c")
        sid = lax.axis_index("s")
        col0 = cid * HHALF
        pltpu.sync_copy(init_hbm.at[pl.ds(sid * RSUB, RSUB), pl.ds(col0, HHALF)],
                        agg_sh.at[pl.ds(sid * RSUB, RSUB)])

        @pl.when(sid == 15)
        def _():
            pltpu.sync_copy(init_hbm.at[pl.ds(16 * RSUB, RTAIL), pl.ds(col0, HHALF)],
                            agg_sh.at[pl.ds(16 * RSUB, RTAIL)])

        plsc.subcore_barrier()
        pltpu.sync_copy(dstr_hbm.at[sid], idx_v)
        ebase = sid * eps

        buf = (buf_v, buf2_v)
        lsem = (ls0, ls1)
        ssem = (ss0, ss1)

        def l_desc(g, slot):
            return pltpu.make_async_copy(
                m_hbm.at[cid, pl.ds(ebase + g * SCH, SCH)],
                buf[slot], lsem[slot])

        def s_desc(g, slot):
            return pltpu.make_async_copy(buf[slot], agg_sh.at[idx_v.at[g]],
                                         ssem[slot])

        l_desc(0, 0).start()

        def phase(g, slot):
            @pl.when(g + 1 < nsc)
            def _():
                @pl.when(g >= 1)
                def _():
                    s_desc(g - 1, 1 - slot).wait()
                l_desc(g + 1, 1 - slot).start()

            l_desc(g, slot).wait()
            s_desc(g, slot).start(add=True)

        @pl.loop(0, nsc // 2)
        def _(i):
            phase(2 * i, 0)
            phase(2 * i + 1, 1)

        if nsc % 2:
            phase(nsc - 1, (nsc - 1) % 2)

        for k in (nsc - 2, nsc - 1):
            s_desc(k, k % 2).wait()

        plsc.subcore_barrier()
        pltpu.sync_copy(agg_sh.at[pl.ds(sid * RSUB, RSUB)],
                        agg_hbm.at[pl.ds(sid * RSUB, RSUB), pl.ds(col0, HHALF)])

        @pl.when(sid == 15)
        def _():
            pltpu.sync_copy(agg_sh.at[pl.ds(16 * RSUB, RTAIL)],
                            agg_hbm.at[pl.ds(16 * RSUB, RTAIL), pl.ds(col0, HHALF)])

    return _scatter_body


def _scalar_body(src_hbm, dst_hbm, s_hbm, p_hbm, part_hbm,
                 src_v, dst_v, s_v, p_v, deg_v, cnt_v, ss_v, sp_v, ssp_v):
    wid = lax.axis_index("s") * 2 + lax.axis_index("c")
    base = wid * EPW
    pltpu.sync_copy(src_hbm.at[pl.ds(base, EPW)], src_v)
    pltpu.sync_copy(dst_hbm.at[pl.ds(base, EPW)], dst_v)
    pltpu.sync_copy(s_hbm.at[pl.ds(base, EPW)], s_v)
    pltpu.sync_copy(p_hbm.at[pl.ds(base, EPW)], p_v)

    zero16 = jnp.zeros((16,), jnp.float32)

    for ref in (deg_v, cnt_v, ss_v, sp_v, ssp_v):
        @pl.loop(0, N // 16)
        def _(i, ref=ref):
            ref[pl.ds(i * 16, 16)] = zero16

    ones = jnp.ones((16,), jnp.float32)

    @pl.loop(0, EPW // 16)
    def _(i):
        off = i * 16
        si = src_v[pl.ds(off, 16)]
        di = dst_v[pl.ds(off, 16)]
        sv = s_v[pl.ds(off, 16)]
        pv = p_v[pl.ds(off, 16)]
        plsc.addupdate_scatter(deg_v, [di], ones)
        plsc.addupdate_scatter(cnt_v, [si], ones)
        plsc.addupdate_scatter(ss_v, [si], sv)
        plsc.addupdate_scatter(sp_v, [si], pv)
        plsc.addupdate_scatter(ssp_v, [si], sv * pv)

    for k, ref in enumerate((deg_v, cnt_v, ss_v, sp_v, ssp_v)):
        pltpu.sync_copy(ref, part_hbm.at[pl.ds(wid * 5 * N + k * N, N)])


def _final_body(h_ref, agg_ref, part_ref, s_ref, p_ref, wc_ref, bc_ref, out_ref):
    parts = jnp.sum(part_ref[...], axis=0)
    deg = jnp.maximum(parts[0], 1.0)
    cnt_raw = parts[1]
    cnt = jnp.maximum(cnt_raw, 1.0)
    ssum = parts[2]
    psum = parts[3]
    spsum = parts[4]

    s = s_ref[...]
    p = p_ref[...]
    smin = jnp.min(s)
    smax = jnp.max(s)
    pmin = jnp.min(p)
    pmax = jnp.max(p)
    rs = smax - smin + 1e-12
    rp = pmax - pmin + 1e-12
    numer = spsum - pmin * ssum - smin * psum + cnt_raw * (smin * pmin)
    node_w = numer / (rs * rp) / cnt

    h2 = h_ref[...] + agg_ref[...] / deg[:, None]
    hw = h2 * (1.0 + node_w)[:, None]

    acc = hw[0:NC]
    for k in range(1, N // NC):
        acc = acc + hw[k * NC:(k + 1) * NC]
    tail = N - (N // NC) * NC
    acc = acc + jnp.concatenate(
        [hw[(N // NC) * NC:N], jnp.zeros((NC - tail, H), jnp.float32)], axis=0)

    cidx = lax.broadcasted_iota(jnp.int32, (NC, 1), 0)
    ccnt = jnp.where(cidx < tail, float(N // NC + 1), float(N // NC))
    sfeat = acc / ccnt
    z = jnp.dot(sfeat, wc_ref[...], preferred_element_type=jnp.float32) + bc_ref[...]
    out_ref[...] = jax.nn.sigmoid(jax.nn.sigmoid(z))


_SC_CALLS = None


def _make_gather_call(mesh, eh):
    epw = eh // NWORK
    ngf = epw // GCH
    gtail = epw - ngf * GCH
    return pl.kernel(
        _make_gather_body(epw, ngf, gtail),
        out_type=(jax.ShapeDtypeStruct((eh, H), jnp.float32),
                  jax.ShapeDtypeStruct((eh, H), jnp.float32)),
        mesh=mesh,
        scratch_types=[
            pltpu.VMEM((epw,), jnp.int32),
            pltpu.VMEM((epw,), jnp.int32),
        ] + [pltpu.VMEM((GCH, H), jnp.float32)] * 6
          + [pltpu.SemaphoreType.DMA] * 12,
    )


def _make_scatter_call(mesh, eh):
    eps = eh // 16
    nsc = eps // SCH
    return pl.kernel(
        _make_scatter_body(eps, nsc),
        out_type=jax.ShapeDtypeStruct((N, H), jnp.float32),
        mesh=mesh,
        scratch_types=[
            pltpu.VMEM((nsc, SCH), jnp.int32),
            pltpu.VMEM((SCH, HHALF), jnp.float32),
            pltpu.VMEM((SCH, HHALF), jnp.float32),
            pltpu.VMEM_SHARED((N, HHALF), jnp.float32),
        ] + [pltpu.SemaphoreType.DMA] * 4,
    )


def _get_sc_calls():
    global _SC_CALLS
    if _SC_CALLS is not None:
        return _SC_CALLS
    mesh = plsc.VectorSubcoreMesh(core_axis_name="c", subcore_axis_name="s")

    gather1 = _make_gather_call(mesh, EH1)
    gather2 = _make_gather_call(mesh, EH2)
    scatter1 = _make_scatter_call(mesh, EH1)
    scatter2 = _make_scatter_call(mesh, EH2)

    scalar_call = pl.kernel(
        _scalar_body,
        out_type=jax.ShapeDtypeStruct((NWORK * 5 * N,), jnp.float32),
        mesh=mesh,
        compiler_params=pltpu.CompilerParams(needs_layout_passes=False),
        scratch_types=[
            pltpu.VMEM((EPW,), jnp.int32),
            pltpu.VMEM((EPW,), jnp.int32),
            pltpu.VMEM((EPW,), jnp.float32),
            pltpu.VMEM((EPW,), jnp.float32),
            pltpu.VMEM((N,), jnp.float32),
            pltpu.VMEM((N,), jnp.float32),
            pltpu.VMEM((N,), jnp.float32),
            pltpu.VMEM((N,), jnp.float32),
            pltpu.VMEM((N,), jnp.float32),
        ],
    )
    _SC_CALLS = (gather1, gather2, scatter1, scatter2, scalar_call)
    return _SC_CALLS

NB = 1000
EB = 3200


def _full2(i):
    return (0, 0)


def _edge_pass(eh, edge_attr, Asrc, Bdst, We, be, WmC, bm, Wsp, bsp):
    return pl.pallas_call(
        _edge_body,
        grid=(eh // EB,),
        in_specs=[
            pl.BlockSpec((EB, DE), lambda i: (i, 0)),
            pl.BlockSpec((EB, H), lambda i: (i, 0)),
            pl.BlockSpec((EB, H), lambda i: (i, 0)),
            pl.BlockSpec((DE, H), _full2), pl.BlockSpec((1, H), _full2),
            pl.BlockSpec((H, H), _full2), pl.BlockSpec((1, H), _full2),
            pl.BlockSpec((H, 2), _full2), pl.BlockSpec((1, 2), _full2),
        ],
        out_specs=[
            pl.BlockSpec((2, EB, HHALF), lambda i: (0, i, 0)),
            pl.BlockSpec((1, 1, EB), lambda i: (i, 0, 0)),
            pl.BlockSpec((1, 1, EB), lambda i: (i, 0, 0)),
        ],
        out_shape=[
            jax.ShapeDtypeStruct((2, eh, HHALF), jnp.float32),
            jax.ShapeDtypeStruct((eh // EB, 1, EB), jnp.float32),
            jax.ShapeDtypeStruct((eh // EB, 1, EB), jnp.float32),
        ],
    )(edge_attr, Asrc, Bdst, We, be, WmC, bm, Wsp, bsp)


@jax.jit
def _run(x, src, dst, edge_attr, Wn1, bn1, Wn2, bn2, We, be,
         WmA, WmB, WmC, bm, Wsp, bsp, Wc, bc, zeros_full):
    gather1, gather2, scatter1, scatter2, scalar_call = _get_sc_calls()
    h, A, B = pl.pallas_call(
        _node_prep_body,
        grid=(N // NB,),
        in_specs=[
            pl.BlockSpec((NB, D), lambda i: (i, 0)),
            pl.BlockSpec((D, H), _full2), pl.BlockSpec((1, H), _full2),
            pl.BlockSpec((H, H), _full2), pl.BlockSpec((1, H), _full2),
            pl.BlockSpec((H, H), _full2), pl.BlockSpec((H, H), _full2),
        ],
        out_specs=[pl.BlockSpec((NB, H), lambda i: (i, 0))] * 3,
        out_shape=[jax.ShapeDtypeStruct((N, H), jnp.float32)] * 3,
    )(x, Wn1, bn1, Wn2, bn2, WmA, WmB)

    src1, src2 = src[:EH1], src[EH1:]
    dst1, dst2 = dst[:EH1], dst[EH1:]
    dstr1 = dst1.reshape(16, (EH1 // 16) // SCH, SCH)
    dstr2 = dst2.reshape(16, (EH2 // 16) // SCH, SCH)

    Asrc1, Bdst1 = gather1(A, B, src1, dst1)
    Asrc2, Bdst2 = gather2(A, B, src2, dst2)

    ea1, ea2 = edge_attr[:EH1], edge_attr[EH1:]
    m1, s1, p1 = _edge_pass(EH1, ea1, Asrc1, Bdst1, We, be, WmC, bm, Wsp, bsp)
    agg1 = scatter1(m1, dstr1, zeros_full)
    m2, s2, p2 = _edge_pass(EH2, ea2, Asrc2, Bdst2, We, be, WmC, bm, Wsp, bsp)
    agg = scatter2(m2, dstr2, agg1)

    s_e = jnp.concatenate([s1.reshape(EH1), s2.reshape(EH2)])
    p_e = jnp.concatenate([p1.reshape(EH1), p2.reshape(EH2)])
    parts = scalar_call(src, dst, s_e, p_e).reshape(NWORK, 5, N)

    out = pl.pallas_call(
        _final_body,
        in_specs=[
            pl.BlockSpec((N, H)),
            pl.BlockSpec((N, H)),
            pl.BlockSpec((NWORK, 5, N)),
            pl.BlockSpec((E,)),
            pl.BlockSpec((E,)),
            pl.BlockSpec((H, 1)),
            pl.BlockSpec((1, 1)),
        ],
        out_specs=pl.BlockSpec((NC, 1)),
        out_shape=jax.ShapeDtypeStruct((NC, 1), jnp.float32),
    )(h, agg, parts, s_e, p_e, Wc, bc)
    return out


def kernel(x, edge_index, edge_attr, Wn1, bn1, Wn2, bn2, We, be, Wm, bm,
           Ws, bs, Wp, bp, Wc, bc):
    src = edge_index[0].astype(jnp.int32)
    dst = edge_index[1].astype(jnp.int32)
    WmA = Wm[:H]
    WmB = Wm[H:2 * H]
    WmC = Wm[2 * H:]
    Wsp = jnp.concatenate([Ws, Wp], axis=1)
    bsp = jnp.concatenate([bs, bp]).reshape(1, 2)
    zeros_full = jnp.zeros((N, H), jnp.float32)
    return _run(x, src, dst, edge_attr,
                Wn1, bn1.reshape(1, H), Wn2, bn2.reshape(1, H),
                We, be.reshape(1, H), WmA, WmB, WmC, bm.reshape(1, H),
                Wsp, bsp, Wc, bc.reshape(1, 1), zeros_full)


# speedup vs baseline: 1.0048x; 1.0048x over previous
import jax
import jax.numpy as jnp
from jax import lax
from jax.experimental import pallas as pl
from jax.experimental.pallas import tpu as pltpu
from jax.experimental.pallas import tpu_sc as plsc

N = 10000
E = 160000
D = 256
DE = 16
H = 256
NC = 1024

NWORK = 32
EPW = E // NWORK
GCH = 72
SCH = 80
RSUB = 624
RTAIL = N - 16 * RSUB
HHALF = H // 2

EH1 = 83200
EH2 = E - EH1


def _node_prep_body(x_ref, wn1_ref, bn1_ref, wn2_ref, bn2_ref, wma_ref, wmb_ref,
                    h_ref, a_ref, b_ref):
    h = jax.nn.relu(jnp.dot(x_ref[...], wn1_ref[...],
                            preferred_element_type=jnp.float32) + bn1_ref[...])
    h = jax.nn.relu(jnp.dot(h, wn2_ref[...],
                            preferred_element_type=jnp.float32) + bn2_ref[...])
    h_ref[...] = h
    a_ref[...] = jnp.dot(h, wma_ref[...], preferred_element_type=jnp.float32)
    b_ref[...] = jnp.dot(h, wmb_ref[...], preferred_element_type=jnp.float32)


def _make_gather_body(epw, ngf, gtail):
    assert ngf % 3 == 0

    def _gather_body(a_hbm, b_hbm, src_hbm, dst_hbm, asrc_hbm, bdst_hbm,
                     sidx_v, didx_v, bufa0, bufb0, bufa1, bufb1, bufa2, bufb2,
                     gsa0, gsb0, gsa1, gsb1, gsa2, gsb2,
                     wsa0, wsb0, wsa1, wsb1, wsa2, wsb2):
        wid = lax.axis_index("s") * 2 + lax.axis_index("c")
        base = wid * epw
        pltpu.sync_copy(src_hbm.at[pl.ds(base, epw)], sidx_v)
        pltpu.sync_copy(dst_hbm.at[pl.ds(base, epw)], didx_v)

        bufa = (bufa0, bufa1, bufa2)
        bufb = (bufb0, bufb1, bufb2)
        gsa = (gsa0, gsa1, gsa2)
        gsb = (gsb0, gsb1, gsb2)
        wsa = (wsa0, wsa1, wsa2)
        wsb = (wsb0, wsb1, wsb2)

        def g_desc(off, slot):
            return (pltpu.make_async_copy(a_hbm.at[sidx_v.at[pl.ds(off, GCH)]],
                                          bufa[slot], gsa[slot]),
                    pltpu.make_async_copy(b_hbm.at[didx_v.at[pl.ds(off, GCH)]],
                                          bufb[slot], gsb[slot]))

        def w_desc(off, slot):
            return (pltpu.make_async_copy(
                        bufa[slot], asrc_hbm.at[pl.ds(base + off, GCH)], wsa[slot]),
                    pltpu.make_async_copy(
                        bufb[slot], bdst_hbm.at[pl.ds(base + off, GCH)], wsb[slot]))

        for d in g_desc(0, 0):
            d.start()
        for d in g_desc(GCH, 1):
            d.start()

        def phase(g, slot):
            off = g * GCH
            nslot = (slot + 2) % 3

            @pl.when(g + 2 < ngf)
            def _():
                @pl.when(g >= 1)
                def _():
                    for d in w_desc(off - GCH, nslot):
                        d.wait()
                for d in g_desc(off + 2 * GCH, nslot):
                    d.start()

            for d in g_desc(off, slot):
                d.wait()
            for d in w_desc(off, slot):
                d.start()

        @pl.loop(0, ngf // 3)
        def _(i):
            phase(3 * i, 0)
            phase(3 * i + 1, 1)
            phase(3 * i + 2, 2)

        for k in (ngf - 3, ngf - 2, ngf - 1):
            for d in w_desc(k * GCH, k % 3):
                d.wait()

        if gtail:
            toff = ngf * GCH
            pltpu.sync_copy(a_hbm.at[sidx_v.at[pl.ds(toff, gtail)]],
                            bufa0.at[pl.ds(0, gtail)])
            pltpu.sync_copy(b_hbm.at[didx_v.at[pl.ds(toff, gtail)]],
                            bufb0.at[pl.ds(0, gtail)])
            pltpu.sync_copy(bufa0.at[pl.ds(0, gtail)],
                            asrc_hbm.at[pl.ds(base + toff, gtail)])
            pltpu.sync_copy(bufb0.at[pl.ds(0, gtail)],
                            bdst_hbm.at[pl.ds(base + toff, gtail)])

    return _gather_body


def _edge_body(ea_ref, asrc_ref, bdst_ref, we_ref, be_ref, wmc_ref, bm_ref,
               wsp_ref, bsp_ref, m_ref, s_ref, p_ref):
    e = jax.nn.relu(jnp.dot(ea_ref[...], we_ref[...],
                            preferred_element_type=jnp.float32) + be_ref[...])
    ec = jnp.dot(e, wmc_ref[...], preferred_element_type=jnp.float32)
    m = jax.nn.relu(asrc_ref[...] + bdst_ref[...] + ec + bm_ref[...])
    m_ref[...] = m
    u = e + m
    sp = jnp.dot(u, wsp_ref[...], preferred_element_type=jnp.float32) + bsp_ref[...]
    s_ref[0, 0, :] = jax.nn.sigmoid(sp[:, 0])
    p_ref[0, 0, :] = sp[:, 1]


def _make_scatter_body(eps, nsc):
    def _scatter_body(m_hbm, dstr_hbm, init_hbm, agg_hbm, idx_v, buf_v, buf2_v,
                      agg_sh, ls0, ls1, ss0, ss1):
        cid = lax.axis_index("c")
        sid = lax.axis_index("s")
        col0 = cid * HHALF
        pltpu.sync_copy(init_hbm.at[pl.ds(sid * RSUB, RSUB), pl.ds(col0, HHALF)],
                        agg_sh.at[pl.ds(sid * RSUB, RSUB)])

        @pl.when(sid == 15)
        def _():
            pltpu.sync_copy(init_hbm.at[pl.ds(16 * RSUB, RTAIL), pl.ds(col0, HHALF)],
                            agg_sh.at[pl.ds(16 * RSUB, RTAIL)])

        plsc.subcore_barrier()
        pltpu.sync_copy(dstr_hbm.at[sid], idx_v)
        ebase = sid * eps

        buf = (buf_v, buf2_v)
        lsem = (ls0, ls1)
        ssem = (ss0, ss1)

        def l_desc(g, slot):
            return pltpu.make_async_copy(
                m_hbm.at[pl.ds(ebase + g * SCH, SCH), pl.ds(col0, HHALF)],
                buf[slot], lsem[slot])

        def s_desc(g, slot):
            return pltpu.make_async_copy(buf[slot], agg_sh.at[idx_v.at[g]],
                                         ssem[slot])

        l_desc(0, 0).start()

        def phase(g, slot):
            @pl.when(g + 1 < nsc)
            def _():
                @pl.when(g >= 1)
                def _():
                    s_desc(g - 1, 1 - slot).wait()
                l_desc(g + 1, 1 - slot).start()

            l_desc(g, slot).wait()
            s_desc(g, slot).start(add=True)

        @pl.loop(0, nsc // 2)
        def _(i):
            phase(2 * i, 0)
            phase(2 * i + 1, 1)

        if nsc % 2:
            phase(nsc - 1, (nsc - 1) % 2)

        for k in (nsc - 2, nsc - 1):
            s_desc(k, k % 2).wait()

        plsc.subcore_barrier()
        pltpu.sync_copy(agg_sh.at[pl.ds(sid * RSUB, RSUB)],
                        agg_hbm.at[pl.ds(sid * RSUB, RSUB), pl.ds(col0, HHALF)])

        @pl.when(sid == 15)
        def _():
            pltpu.sync_copy(agg_sh.at[pl.ds(16 * RSUB, RTAIL)],
                            agg_hbm.at[pl.ds(16 * RSUB, RTAIL), pl.ds(col0, HHALF)])

    return _scatter_body


def _scalar_body(src_hbm, dst_hbm, s_hbm, p_hbm, part_hbm,
                 src_v, dst_v, s_v, p_v, deg_v, cnt_v, ss_v, sp_v, ssp_v):
    wid = lax.axis_index("s") * 2 + lax.axis_index("c")
    base = wid * EPW
    pltpu.sync_copy(src_hbm.at[pl.ds(base, EPW)], src_v)
    pltpu.sync_copy(dst_hbm.at[pl.ds(base, EPW)], dst_v)
    pltpu.sync_copy(s_hbm.at[pl.ds(base, EPW)], s_v)
    pltpu.sync_copy(p_hbm.at[pl.ds(base, EPW)], p_v)

    zero16 = jnp.zeros((16,), jnp.float32)

    for ref in (deg_v, cnt_v, ss_v, sp_v, ssp_v):
        @pl.loop(0, N // 16)
        def _(i, ref=ref):
            ref[pl.ds(i * 16, 16)] = zero16

    ones = jnp.ones((16,), jnp.float32)

    @pl.loop(0, EPW // 16)
    def _(i):
        off = i * 16
        si = src_v[pl.ds(off, 16)]
        di = dst_v[pl.ds(off, 16)]
        sv = s_v[pl.ds(off, 16)]
        pv = p_v[pl.ds(off, 16)]
        plsc.addupdate_scatter(deg_v, [di], ones)
        plsc.addupdate_scatter(cnt_v, [si], ones)
        plsc.addupdate_scatter(ss_v, [si], sv)
        plsc.addupdate_scatter(sp_v, [si], pv)
        plsc.addupdate_scatter(ssp_v, [si], sv * pv)

    for k, ref in enumerate((deg_v, cnt_v, ss_v, sp_v, ssp_v)):
        pltpu.sync_copy(ref, part_hbm.at[pl.ds(wid * 5 * N + k * N, N)])


def _final_body(h_ref, agg_ref, part_ref, s_ref, p_ref, wc_ref, bc_ref, out_ref):
    parts = jnp.sum(part_ref[...], axis=0)
    deg = jnp.maximum(parts[0], 1.0)
    cnt_raw = parts[1]
    cnt = jnp.maximum(cnt_raw, 1.0)
    ssum = parts[2]
    psum = parts[3]
    spsum = parts[4]

    s = s_ref[...]
    p = p_ref[...]
    smin = jnp.min(s)
    smax = jnp.max(s)
    pmin = jnp.min(p)
    pmax = jnp.max(p)
    rs = smax - smin + 1e-12
    rp = pmax - pmin + 1e-12
    numer = spsum - pmin * ssum - smin * psum + cnt_raw * (smin * pmin)
    node_w = numer / (rs * rp) / cnt

    h2 = h_ref[...] + agg_ref[...] / deg[:, None]
    hw = h2 * (1.0 + node_w)[:, None]

    acc = hw[0:NC]
    for k in range(1, N // NC):
        acc = acc + hw[k * NC:(k + 1) * NC]
    tail = N - (N // NC) * NC
    acc = acc + jnp.concatenate(
        [hw[(N // NC) * NC:N], jnp.zeros((NC - tail, H), jnp.float32)], axis=0)

    cidx = lax.broadcasted_iota(jnp.int32, (NC, 1), 0)
    ccnt = jnp.where(cidx < tail, float(N // NC + 1), float(N // NC))
    sfeat = acc / ccnt
    z = jnp.dot(sfeat, wc_ref[...], preferred_element_type=jnp.float32) + bc_ref[...]
    out_ref[...] = jax.nn.sigmoid(jax.nn.sigmoid(z))


_SC_CALLS = None


def _make_gather_call(mesh, eh):
    epw = eh // NWORK
    ngf = epw // GCH
    gtail = epw - ngf * GCH
    return pl.kernel(
        _make_gather_body(epw, ngf, gtail),
        out_type=(jax.ShapeDtypeStruct((eh, H), jnp.float32),
                  jax.ShapeDtypeStruct((eh, H), jnp.float32)),
        mesh=mesh,
        scratch_types=[
            pltpu.VMEM((epw,), jnp.int32),
            pltpu.VMEM((epw,), jnp.int32),
        ] + [pltpu.VMEM((GCH, H), jnp.float32)] * 6
          + [pltpu.SemaphoreType.DMA] * 12,
    )


def _make_scatter_call(mesh, eh):
    eps = eh // 16
    nsc = eps // SCH
    return pl.kernel(
        _make_scatter_body(eps, nsc),
        out_type=jax.ShapeDtypeStruct((N, H), jnp.float32),
        mesh=mesh,
        scratch_types=[
            pltpu.VMEM((nsc, SCH), jnp.int32),
            pltpu.VMEM((SCH, HHALF), jnp.float32),
            pltpu.VMEM((SCH, HHALF), jnp.float32),
            pltpu.VMEM_SHARED((N, HHALF), jnp.float32),
        ] + [pltpu.SemaphoreType.DMA] * 4,
    )


def _get_sc_calls():
    global _SC_CALLS
    if _SC_CALLS is not None:
        return _SC_CALLS
    mesh = plsc.VectorSubcoreMesh(core_axis_name="c", subcore_axis_name="s")

    gather1 = _make_gather_call(mesh, EH1)
    gather2 = _make_gather_call(mesh, EH2)
    scatter1 = _make_scatter_call(mesh, EH1)
    scatter2 = _make_scatter_call(mesh, EH2)

    scalar_call = pl.kernel(
        _scalar_body,
        out_type=jax.ShapeDtypeStruct((NWORK * 5 * N,), jnp.float32),
        mesh=mesh,
        compiler_params=pltpu.CompilerParams(needs_layout_passes=False),
        scratch_types=[
            pltpu.VMEM((EPW,), jnp.int32),
            pltpu.VMEM((EPW,), jnp.int32),
            pltpu.VMEM((EPW,), jnp.float32),
            pltpu.VMEM((EPW,), jnp.float32),
            pltpu.VMEM((N,), jnp.float32),
            pltpu.VMEM((N,), jnp.float32),
            pltpu.VMEM((N,), jnp.float32),
            pltpu.VMEM((N,), jnp.float32),
            pltpu.VMEM((N,), jnp.float32),
        ],
    )
    _SC_CALLS = (gather1, gather2, scatter1, scatter2, scalar_call)
    return _SC_CALLS

NB = 1000
EB = 3200


def _full2(i):
    return (0, 0)


def _edge_pass(eh, edge_attr, Asrc, Bdst, We, be, WmC, bm, Wsp, bsp):
    return pl.pallas_call(
        _edge_body,
        grid=(eh // EB,),
        in_specs=[
            pl.BlockSpec((EB, DE), lambda i: (i, 0)),
            pl.BlockSpec((EB, H), lambda i: (i, 0)),
            pl.BlockSpec((EB, H), lambda i: (i, 0)),
            pl.BlockSpec((DE, H), _full2), pl.BlockSpec((1, H), _full2),
            pl.BlockSpec((H, H), _full2), pl.BlockSpec((1, H), _full2),
            pl.BlockSpec((H, 2), _full2), pl.BlockSpec((1, 2), _full2),
        ],
        out_specs=[
            pl.BlockSpec((EB, H), lambda i: (i, 0)),
            pl.BlockSpec((1, 1, EB), lambda i: (i, 0, 0)),
            pl.BlockSpec((1, 1, EB), lambda i: (i, 0, 0)),
        ],
        out_shape=[
            jax.ShapeDtypeStruct((eh, H), jnp.float32),
            jax.ShapeDtypeStruct((eh // EB, 1, EB), jnp.float32),
            jax.ShapeDtypeStruct((eh // EB, 1, EB), jnp.float32),
        ],
    )(edge_attr, Asrc, Bdst, We, be, WmC, bm, Wsp, bsp)


@jax.jit
def _run(x, src, dst, edge_attr, Wn1, bn1, Wn2, bn2, We, be,
         WmA, WmB, WmC, bm, Wsp, bsp, Wc, bc, zeros_full):
    gather1, gather2, scatter1, scatter2, scalar_call = _get_sc_calls()
    h, A, B = pl.pallas_call(
        _node_prep_body,
        grid=(N // NB,),
        in_specs=[
            pl.BlockSpec((NB, D), lambda i: (i, 0)),
            pl.BlockSpec((D, H), _full2), pl.BlockSpec((1, H), _full2),
            pl.BlockSpec((H, H), _full2), pl.BlockSpec((1, H), _full2),
            pl.BlockSpec((H, H), _full2), pl.BlockSpec((H, H), _full2),
        ],
        out_specs=[pl.BlockSpec((NB, H), lambda i: (i, 0))] * 3,
        out_shape=[jax.ShapeDtypeStruct((N, H), jnp.float32)] * 3,
    )(x, Wn1, bn1, Wn2, bn2, WmA, WmB)

    src1, src2 = src[:EH1], src[EH1:]
    dst1, dst2 = dst[:EH1], dst[EH1:]
    dstr1 = dst1.reshape(16, (EH1 // 16) // SCH, SCH)
    dstr2 = dst2.reshape(16, (EH2 // 16) // SCH, SCH)

    Asrc1, Bdst1 = gather1(A, B, src1, dst1)
    Asrc2, Bdst2 = gather2(A, B, src2, dst2)

    ea1, ea2 = edge_attr[:EH1], edge_attr[EH1:]
    m1, s1, p1 = _edge_pass(EH1, ea1, Asrc1, Bdst1, We, be, WmC, bm, Wsp, bsp)
    agg1 = scatter1(m1, dstr1, zeros_full)
    m2, s2, p2 = _edge_pass(EH2, ea2, Asrc2, Bdst2, We, be, WmC, bm, Wsp, bsp)
    agg = scatter2(m2, dstr2, agg1)

    s_e = jnp.concatenate([s1.reshape(EH1), s2.reshape(EH2)])
    p_e = jnp.concatenate([p1.reshape(EH1), p2.reshape(EH2)])
    parts = scalar_call(src, dst, s_e, p_e).reshape(NWORK, 5, N)

    out = pl.pallas_call(
        _final_body,
        in_specs=[
            pl.BlockSpec((N, H)),
            pl.BlockSpec((N, H)),
            pl.BlockSpec((NWORK, 5, N)),
            pl.BlockSpec((E,)),
            pl.BlockSpec((E,)),
            pl.BlockSpec((H, 1)),
            pl.BlockSpec((1, 1)),
        ],
        out_specs=pl.BlockSpec((NC, 1)),
        out_shape=jax.ShapeDtypeStruct((NC, 1), jnp.float32),
    )(h, agg, parts, s_e, p_e, Wc, bc)
    return out


def kernel(x, edge_index, edge_attr, Wn1, bn1, Wn2, bn2, We, be, Wm, bm,
           Ws, bs, Wp, bp, Wc, bc):
    src = edge_index[0].astype(jnp.int32)
    dst = edge_index[1].astype(jnp.int32)
    WmA = Wm[:H]
    WmB = Wm[H:2 * H]
    WmC = Wm[2 * H:]
    Wsp = jnp.concatenate([Ws, Wp], axis=1)
    bsp = jnp.concatenate([bs, bp]).reshape(1, 2)
    zeros_full = jnp.zeros((N, H), jnp.float32)
    return _run(x, src, dst, edge_attr,
                Wn1, bn1.reshape(1, H), Wn2, bn2.reshape(1, H),
                We, be.reshape(1, H), WmA, WmB, WmC, bm.reshape(1, H),
                Wsp, bsp, Wc, bc.reshape(1, 1), zeros_full)


# speedup vs baseline: 1.1952x; 1.1895x over previous
import jax
import jax.numpy as jnp
from jax import lax
from jax.experimental import pallas as pl
from jax.experimental.pallas import tpu as pltpu
from jax.experimental.pallas import tpu_sc as plsc

N = 10000
E = 160000
D = 256
DE = 16
H = 256
NC = 1024

NWORK = 32
EPW = E // NWORK
GCH = 72
SCH = 80
RSUB = 624
RTAIL = N - 16 * RSUB
HHALF = H // 2

EH1 = 83200
EH2 = E - EH1


def _node_prep_body(x_ref, wn1_ref, bn1_ref, wn2_ref, bn2_ref, wma_ref, wmb_ref,
                    h_ref, a_ref, b_ref):
    h = jax.nn.relu(jnp.dot(x_ref[...], wn1_ref[...],
                            preferred_element_type=jnp.float32) + bn1_ref[...])
    h = jax.nn.relu(jnp.dot(h, wn2_ref[...],
                            preferred_element_type=jnp.float32) + bn2_ref[...])
    h_ref[...] = h
    a_ref[...] = jnp.dot(h, wma_ref[...], preferred_element_type=jnp.float32)
    b_ref[...] = jnp.dot(h, wmb_ref[...], preferred_element_type=jnp.float32)


def _make_gather_body(eps, ngf, gtail):
    assert ngf % 3 == 0

    def _gather_body(a_hbm, b_hbm, src_hbm, dst_hbm, asrc_hbm, bdst_hbm,
                     sidx_v, didx_v, buf0, buf1, buf2, table_sh,
                     gs0, gs1, gs2, ws0, ws1, ws2):
        cid = lax.axis_index("c")
        sid = lax.axis_index("s")
        col0 = cid * HHALF
        base = sid * eps
        pltpu.sync_copy(src_hbm.at[pl.ds(base, eps)], sidx_v)
        pltpu.sync_copy(dst_hbm.at[pl.ds(base, eps)], didx_v)

        buf = (buf0, buf1, buf2)
        gsem = (gs0, gs1, gs2)
        wsem = (ws0, ws1, ws2)

        def stage(tab_hbm):
            pltpu.sync_copy(tab_hbm.at[pl.ds(sid * RSUB, RSUB), pl.ds(col0, HHALF)],
                            table_sh.at[pl.ds(sid * RSUB, RSUB)])

            @pl.when(sid == 15)
            def _():
                pltpu.sync_copy(
                    tab_hbm.at[pl.ds(16 * RSUB, RTAIL), pl.ds(col0, HHALF)],
                    table_sh.at[pl.ds(16 * RSUB, RTAIL)])

            plsc.subcore_barrier()

        def one_pass(idx_v, out_hbm):
            def g_desc(off, slot):
                return pltpu.make_async_copy(
                    table_sh.at[idx_v.at[pl.ds(off, GCH)]], buf[slot], gsem[slot])

            def w_desc(off, slot):
                return pltpu.make_async_copy(
                    buf[slot],
                    out_hbm.at[pl.ds(base + off, GCH), pl.ds(col0, HHALF)],
                    wsem[slot])

            g_desc(0, 0).start()
            g_desc(GCH, 1).start()

            def phase(g, slot):
                off = g * GCH
                nslot = (slot + 2) % 3

                @pl.when(g + 2 < ngf)
                def _():
                    @pl.when(g >= 1)
                    def _():
                        w_desc(off - GCH, nslot).wait()
                    g_desc(off + 2 * GCH, nslot).start()

                g_desc(off, slot).wait()
                w_desc(off, slot).start()

            @pl.loop(0, ngf // 3)
            def _(i):
                phase(3 * i, 0)
                phase(3 * i + 1, 1)
                phase(3 * i + 2, 2)

            for k in (ngf - 3, ngf - 2, ngf - 1):
                w_desc(k * GCH, k % 3).wait()

            if gtail:
                toff = ngf * GCH
                pltpu.sync_copy(table_sh.at[idx_v.at[pl.ds(toff, gtail)]],
                                buf0.at[pl.ds(0, gtail)])
                pltpu.sync_copy(
                    buf0.at[pl.ds(0, gtail)],
                    out_hbm.at[pl.ds(base + toff, gtail), pl.ds(col0, HHALF)])

        stage(a_hbm)
        one_pass(sidx_v, asrc_hbm)
        plsc.subcore_barrier()
        stage(b_hbm)
        one_pass(didx_v, bdst_hbm)

    return _gather_body


def _edge_body(ea_ref, asrc_ref, bdst_ref, we_ref, be_ref, wmc_ref, bm_ref,
               wsp_ref, bsp_ref, m_ref, s_ref, p_ref):
    e = jax.nn.relu(jnp.dot(ea_ref[...], we_ref[...],
                            preferred_element_type=jnp.float32) + be_ref[...])
    ec = jnp.dot(e, wmc_ref[...], preferred_element_type=jnp.float32)
    m = jax.nn.relu(asrc_ref[...] + bdst_ref[...] + ec + bm_ref[...])
    m_ref[...] = m
    u = e + m
    sp = jnp.dot(u, wsp_ref[...], preferred_element_type=jnp.float32) + bsp_ref[...]
    s_ref[0, 0, :] = jax.nn.sigmoid(sp[:, 0])
    p_ref[0, 0, :] = sp[:, 1]


def _make_scatter_body(eps, nsc):
    def _scatter_body(m_hbm, dstr_hbm, init_hbm, agg_hbm, idx_v, buf_v, buf2_v,
                      agg_sh, ls0, ls1, ss0, ss1):
        cid = lax.axis_index("c")
        sid = lax.axis_index("s")
        col0 = cid * HHALF
        pltpu.sync_copy(init_hbm.at[pl.ds(sid * RSUB, RSUB), pl.ds(col0, HHALF)],
                        agg_sh.at[pl.ds(sid * RSUB, RSUB)])

        @pl.when(sid == 15)
        def _():
            pltpu.sync_copy(init_hbm.at[pl.ds(16 * RSUB, RTAIL), pl.ds(col0, HHALF)],
                            agg_sh.at[pl.ds(16 * RSUB, RTAIL)])

        plsc.subcore_barrier()
        pltpu.sync_copy(dstr_hbm.at[sid], idx_v)
        ebase = sid * eps

        buf = (buf_v, buf2_v)
        lsem = (ls0, ls1)
        ssem = (ss0, ss1)

        def l_desc(g, slot):
            return pltpu.make_async_copy(
                m_hbm.at[pl.ds(ebase + g * SCH, SCH), pl.ds(col0, HHALF)],
                buf[slot], lsem[slot])

        def s_desc(g, slot):
            return pltpu.make_async_copy(buf[slot], agg_sh.at[idx_v.at[g]],
                                         ssem[slot])

        l_desc(0, 0).start()

        def phase(g, slot):
            @pl.when(g + 1 < nsc)
            def _():
                @pl.when(g >= 1)
                def _():
                    s_desc(g - 1, 1 - slot).wait()
                l_desc(g + 1, 1 - slot).start()

            l_desc(g, slot).wait()
            s_desc(g, slot).start(add=True)

        @pl.loop(0, nsc // 2)
        def _(i):
            phase(2 * i, 0)
            phase(2 * i + 1, 1)

        if nsc % 2:
            phase(nsc - 1, (nsc - 1) % 2)

        for k in (nsc - 2, nsc - 1):
            s_desc(k, k % 2).wait()

        plsc.subcore_barrier()
        pltpu.sync_copy(agg_sh.at[pl.ds(sid * RSUB, RSUB)],
                        agg_hbm.at[pl.ds(sid * RSUB, RSUB), pl.ds(col0, HHALF)])

        @pl.when(sid == 15)
        def _():
            pltpu.sync_copy(agg_sh.at[pl.ds(16 * RSUB, RTAIL)],
                            agg_hbm.at[pl.ds(16 * RSUB, RTAIL), pl.ds(col0, HHALF)])

    return _scatter_body


def _scalar_body(src_hbm, dst_hbm, s_hbm, p_hbm, part_hbm,
                 src_v, dst_v, s_v, p_v, deg_v, cnt_v, ss_v, sp_v, ssp_v):
    wid = lax.axis_index("s") * 2 + lax.axis_index("c")
    base = wid * EPW
    pltpu.sync_copy(src_hbm.at[pl.ds(base, EPW)], src_v)
    pltpu.sync_copy(dst_hbm.at[pl.ds(base, EPW)], dst_v)
    pltpu.sync_copy(s_hbm.at[pl.ds(base, EPW)], s_v)
    pltpu.sync_copy(p_hbm.at[pl.ds(base, EPW)], p_v)

    zero16 = jnp.zeros((16,), jnp.float32)

    for ref in (deg_v, cnt_v, ss_v, sp_v, ssp_v):
        @pl.loop(0, N // 16)
        def _(i, ref=ref):
            ref[pl.ds(i * 16, 16)] = zero16

    ones = jnp.ones((16,), jnp.float32)

    @pl.loop(0, EPW // 16)
    def _(i):
        off = i * 16
        si = src_v[pl.ds(off, 16)]
        di = dst_v[pl.ds(off, 16)]
        sv = s_v[pl.ds(off, 16)]
        pv = p_v[pl.ds(off, 16)]
        plsc.addupdate_scatter(deg_v, [di], ones)
        plsc.addupdate_scatter(cnt_v, [si], ones)
        plsc.addupdate_scatter(ss_v, [si], sv)
        plsc.addupdate_scatter(sp_v, [si], pv)
        plsc.addupdate_scatter(ssp_v, [si], sv * pv)

    for k, ref in enumerate((deg_v, cnt_v, ss_v, sp_v, ssp_v)):
        pltpu.sync_copy(ref, part_hbm.at[pl.ds(wid * 5 * N + k * N, N)])


def _final_body(h_ref, agg_ref, part_ref, s_ref, p_ref, wc_ref, bc_ref, out_ref):
    parts = jnp.sum(part_ref[...], axis=0)
    deg = jnp.maximum(parts[0], 1.0)
    cnt_raw = parts[1]
    cnt = jnp.maximum(cnt_raw, 1.0)
    ssum = parts[2]
    psum = parts[3]
    spsum = parts[4]

    s = s_ref[...]
    p = p_ref[...]
    smin = jnp.min(s)
    smax = jnp.max(s)
    pmin = jnp.min(p)
    pmax = jnp.max(p)
    rs = smax - smin + 1e-12
    rp = pmax - pmin + 1e-12
    numer = spsum - pmin * ssum - smin * psum + cnt_raw * (smin * pmin)
    node_w = numer / (rs * rp) / cnt

    h2 = h_ref[...] + agg_ref[...] / deg[:, None]
    hw = h2 * (1.0 + node_w)[:, None]

    acc = hw[0:NC]
    for k in range(1, N // NC):
        acc = acc + hw[k * NC:(k + 1) * NC]
    tail = N - (N // NC) * NC
    acc = acc + jnp.concatenate(
        [hw[(N // NC) * NC:N], jnp.zeros((NC - tail, H), jnp.float32)], axis=0)

    cidx = lax.broadcasted_iota(jnp.int32, (NC, 1), 0)
    ccnt = jnp.where(cidx < tail, float(N // NC + 1), float(N // NC))
    sfeat = acc / ccnt
    z = jnp.dot(sfeat, wc_ref[...], preferred_element_type=jnp.float32) + bc_ref[...]
    out_ref[...] = jax.nn.sigmoid(jax.nn.sigmoid(z))


_SC_CALLS = None


def _make_gather_call(mesh, eh):
    eps = eh // 16
    ngf = eps // GCH
    gtail = eps - ngf * GCH
    return pl.kernel(
        _make_gather_body(eps, ngf, gtail),
        out_type=(jax.ShapeDtypeStruct((eh, H), jnp.float32),
                  jax.ShapeDtypeStruct((eh, H), jnp.float32)),
        mesh=mesh,
        scratch_types=[
            pltpu.VMEM((eps,), jnp.int32),
            pltpu.VMEM((eps,), jnp.int32),
        ] + [pltpu.VMEM((GCH, HHALF), jnp.float32)] * 3
          + [pltpu.VMEM_SHARED((N, HHALF), jnp.float32)]
          + [pltpu.SemaphoreType.DMA] * 6,
    )


def _make_scatter_call(mesh, eh):
    eps = eh // 16
    nsc = eps // SCH
    return pl.kernel(
        _make_scatter_body(eps, nsc),
        out_type=jax.ShapeDtypeStruct((N, H), jnp.float32),
        mesh=mesh,
        scratch_types=[
            pltpu.VMEM((nsc, SCH), jnp.int32),
            pltpu.VMEM((SCH, HHALF), jnp.float32),
            pltpu.VMEM((SCH, HHALF), jnp.float32),
            pltpu.VMEM_SHARED((N, HHALF), jnp.float32),
        ] + [pltpu.SemaphoreType.DMA] * 4,
    )


def _get_sc_calls():
    global _SC_CALLS
    if _SC_CALLS is not None:
        return _SC_CALLS
    mesh = plsc.VectorSubcoreMesh(core_axis_name="c", subcore_axis_name="s")

    gather1 = _make_gather_call(mesh, EH1)
    gather2 = _make_gather_call(mesh, EH2)
    scatter1 = _make_scatter_call(mesh, EH1)
    scatter2 = _make_scatter_call(mesh, EH2)

    scalar_call = pl.kernel(
        _scalar_body,
        out_type=jax.ShapeDtypeStruct((NWORK * 5 * N,), jnp.float32),
        mesh=mesh,
        compiler_params=pltpu.CompilerParams(needs_layout_passes=False),
        scratch_types=[
            pltpu.VMEM((EPW,), jnp.int32),
            pltpu.VMEM((EPW,), jnp.int32),
            pltpu.VMEM((EPW,), jnp.float32),
            pltpu.VMEM((EPW,), jnp.float32),
            pltpu.VMEM((N,), jnp.float32),
            pltpu.VMEM((N,), jnp.float32),
            pltpu.VMEM((N,), jnp.float32),
            pltpu.VMEM((N,), jnp.float32),
            pltpu.VMEM((N,), jnp.float32),
        ],
    )
    _SC_CALLS = (gather1, gather2, scatter1, scatter2, scalar_call)
    return _SC_CALLS

NB = 1000
EB = 3200


def _full2(i):
    return (0, 0)


def _edge_pass(eh, edge_attr, Asrc, Bdst, We, be, WmC, bm, Wsp, bsp):
    return pl.pallas_call(
        _edge_body,
        grid=(eh // EB,),
        in_specs=[
            pl.BlockSpec((EB, DE), lambda i: (i, 0)),
            pl.BlockSpec((EB, H), lambda i: (i, 0)),
            pl.BlockSpec((EB, H), lambda i: (i, 0)),
            pl.BlockSpec((DE, H), _full2), pl.BlockSpec((1, H), _full2),
            pl.BlockSpec((H, H), _full2), pl.BlockSpec((1, H), _full2),
            pl.BlockSpec((H, 2), _full2), pl.BlockSpec((1, 2), _full2),
        ],
        out_specs=[
            pl.BlockSpec((EB, H), lambda i: (i, 0)),
            pl.BlockSpec((1, 1, EB), lambda i: (i, 0, 0)),
            pl.BlockSpec((1, 1, EB), lambda i: (i, 0, 0)),
        ],
        out_shape=[
            jax.ShapeDtypeStruct((eh, H), jnp.float32),
            jax.ShapeDtypeStruct((eh // EB, 1, EB), jnp.float32),
            jax.ShapeDtypeStruct((eh // EB, 1, EB), jnp.float32),
        ],
    )(edge_attr, Asrc, Bdst, We, be, WmC, bm, Wsp, bsp)


@jax.jit
def _run(x, src, dst, edge_attr, Wn1, bn1, Wn2, bn2, We, be,
         WmA, WmB, WmC, bm, Wsp, bsp, Wc, bc, zeros_full):
    gather1, gather2, scatter1, scatter2, scalar_call = _get_sc_calls()
    h, A, B = pl.pallas_call(
        _node_prep_body,
        grid=(N // NB,),
        in_specs=[
            pl.BlockSpec((NB, D), lambda i: (i, 0)),
            pl.BlockSpec((D, H), _full2), pl.BlockSpec((1, H), _full2),
            pl.BlockSpec((H, H), _full2), pl.BlockSpec((1, H), _full2),
            pl.BlockSpec((H, H), _full2), pl.BlockSpec((H, H), _full2),
        ],
        out_specs=[pl.BlockSpec((NB, H), lambda i: (i, 0))] * 3,
        out_shape=[jax.ShapeDtypeStruct((N, H), jnp.float32)] * 3,
    )(x, Wn1, bn1, Wn2, bn2, WmA, WmB)

    src1, src2 = src[:EH1], src[EH1:]
    dst1, dst2 = dst[:EH1], dst[EH1:]
    dstr1 = dst1.reshape(16, (EH1 // 16) // SCH, SCH)
    dstr2 = dst2.reshape(16, (EH2 // 16) // SCH, SCH)

    Asrc1, Bdst1 = gather1(A, B, src1, dst1)
    Asrc2, Bdst2 = gather2(A, B, src2, dst2)

    ea1, ea2 = edge_attr[:EH1], edge_attr[EH1:]
    m1, s1, p1 = _edge_pass(EH1, ea1, Asrc1, Bdst1, We, be, WmC, bm, Wsp, bsp)
    agg1 = scatter1(m1, dstr1, zeros_full)
    m2, s2, p2 = _edge_pass(EH2, ea2, Asrc2, Bdst2, We, be, WmC, bm, Wsp, bsp)
    agg = scatter2(m2, dstr2, agg1)

    s_e = jnp.concatenate([s1.reshape(EH1), s2.reshape(EH2)])
    p_e = jnp.concatenate([p1.reshape(EH1), p2.reshape(EH2)])
    parts = scalar_call(src, dst, s_e, p_e).reshape(NWORK, 5, N)

    out = pl.pallas_call(
        _final_body,
        in_specs=[
            pl.BlockSpec((N, H)),
            pl.BlockSpec((N, H)),
            pl.BlockSpec((NWORK, 5, N)),
            pl.BlockSpec((E,)),
            pl.BlockSpec((E,)),
            pl.BlockSpec((H, 1)),
            pl.BlockSpec((1, 1)),
        ],
        out_specs=pl.BlockSpec((NC, 1)),
        out_shape=jax.ShapeDtypeStruct((NC, 1), jnp.float32),
    )(h, agg, parts, s_e, p_e, Wc, bc)
    return out


def kernel(x, edge_index, edge_attr, Wn1, bn1, Wn2, bn2, We, be, Wm, bm,
           Ws, bs, Wp, bp, Wc, bc):
    src = edge_index[0].astype(jnp.int32)
    dst = edge_index[1].astype(jnp.int32)
    WmA = Wm[:H]
    WmB = Wm[H:2 * H]
    WmC = Wm[2 * H:]
    Wsp = jnp.concatenate([Ws, Wp], axis=1)
    bsp = jnp.concatenate([bs, bp]).reshape(1, 2)
    zeros_full = jnp.zeros((N, H), jnp.float32)
    return _run(x, src, dst, edge_attr,
                Wn1, bn1.reshape(1, H), Wn2, bn2.reshape(1, H),
                We, be.reshape(1, H), WmA, WmB, WmC, bm.reshape(1, H),
                Wsp, bsp, Wc, bc.reshape(1, 1), zeros_full)


# speedup vs baseline: 1.1952x; 1.0000x over previous
import jax
import jax.numpy as jnp
from jax import lax
from jax.experimental import pallas as pl
from jax.experimental.pallas import tpu as pltpu
from jax.experimental.pallas import tpu_sc as plsc

N = 10000
E = 160000
D = 256
DE = 16
H = 256
NC = 1024

NWORK = 32
EPW = E // NWORK
GCH = 80
SCH = 80
RSUB = 624
RTAIL = N - 16 * RSUB
HHALF = H // 2

EH1 = 83200
EH2 = E - EH1


def _node_prep_body(x_ref, wn1_ref, bn1_ref, wn2_ref, bn2_ref, wma_ref, wmb_ref,
                    h_ref, a_ref, b_ref):
    h = jax.nn.relu(jnp.dot(x_ref[...], wn1_ref[...],
                            preferred_element_type=jnp.float32) + bn1_ref[...])
    h = jax.nn.relu(jnp.dot(h, wn2_ref[...],
                            preferred_element_type=jnp.float32) + bn2_ref[...])
    h_ref[...] = h
    a_ref[...] = jnp.dot(h, wma_ref[...], preferred_element_type=jnp.float32)
    b_ref[...] = jnp.dot(h, wmb_ref[...], preferred_element_type=jnp.float32)


def _make_gather_body(eps, ngf, gtail):

    def _gather_body(a_hbm, b_hbm, src_hbm, dst_hbm, asrc_hbm, bdst_hbm,
                     sidx_v, didx_v, buf0, buf1, buf2, table_sh,
                     gs0, gs1, gs2, ws0, ws1, ws2):
        cid = lax.axis_index("c")
        sid = lax.axis_index("s")
        col0 = cid * HHALF
        base = sid * eps
        pltpu.sync_copy(src_hbm.at[pl.ds(base, eps)], sidx_v)
        pltpu.sync_copy(dst_hbm.at[pl.ds(base, eps)], didx_v)

        buf = (buf0, buf1, buf2)
        gsem = (gs0, gs1, gs2)
        wsem = (ws0, ws1, ws2)

        def stage(tab_hbm):
            pltpu.sync_copy(tab_hbm.at[pl.ds(sid * RSUB, RSUB), pl.ds(col0, HHALF)],
                            table_sh.at[pl.ds(sid * RSUB, RSUB)])

            @pl.when(sid == 15)
            def _():
                pltpu.sync_copy(
                    tab_hbm.at[pl.ds(16 * RSUB, RTAIL), pl.ds(col0, HHALF)],
                    table_sh.at[pl.ds(16 * RSUB, RTAIL)])

            plsc.subcore_barrier()

        def one_pass(idx_v, out_hbm):
            def g_desc(off, slot):
                return pltpu.make_async_copy(
                    table_sh.at[idx_v.at[pl.ds(off, GCH)]], buf[slot], gsem[slot])

            def w_desc(off, slot):
                return pltpu.make_async_copy(
                    buf[slot],
                    out_hbm.at[pl.ds(base + off, GCH), pl.ds(col0, HHALF)],
                    wsem[slot])

            g_desc(0, 0).start()
            g_desc(GCH, 1).start()

            def phase(g, slot):
                off = g * GCH
                nslot = (slot + 2) % 3

                @pl.when(g + 2 < ngf)
                def _():
                    @pl.when(g >= 1)
                    def _():
                        w_desc(off - GCH, nslot).wait()
                    g_desc(off + 2 * GCH, nslot).start()

                g_desc(off, slot).wait()
                w_desc(off, slot).start()

            @pl.loop(0, ngf // 3)
            def _(i):
                phase(3 * i, 0)
                phase(3 * i + 1, 1)
                phase(3 * i + 2, 2)

            for j in range(ngf - ngf % 3, ngf):
                phase(j, j % 3)

            for k in (ngf - 3, ngf - 2, ngf - 1):
                w_desc(k * GCH, k % 3).wait()

            if gtail:
                toff = ngf * GCH
                pltpu.sync_copy(table_sh.at[idx_v.at[pl.ds(toff, gtail)]],
                                buf0.at[pl.ds(0, gtail)])
                pltpu.sync_copy(
                    buf0.at[pl.ds(0, gtail)],
                    out_hbm.at[pl.ds(base + toff, gtail), pl.ds(col0, HHALF)])

        stage(a_hbm)
        one_pass(sidx_v, asrc_hbm)
        plsc.subcore_barrier()
        stage(b_hbm)
        one_pass(didx_v, bdst_hbm)

    return _gather_body


def _edge_body(ea_ref, asrc_ref, bdst_ref, we_ref, be_ref, wmc_ref, bm_ref,
               wsp_ref, bsp_ref, m_ref, s_ref, p_ref):
    e = jax.nn.relu(jnp.dot(ea_ref[...], we_ref[...],
                            preferred_element_type=jnp.float32) + be_ref[...])
    ec = jnp.dot(e, wmc_ref[...], preferred_element_type=jnp.float32)
    m = jax.nn.relu(asrc_ref[...] + bdst_ref[...] + ec + bm_ref[...])
    m_ref[...] = m
    u = e + m
    sp = jnp.dot(u, wsp_ref[...], preferred_element_type=jnp.float32) + bsp_ref[...]
    s_ref[0, 0, :] = jax.nn.sigmoid(sp[:, 0])
    p_ref[0, 0, :] = sp[:, 1]


def _make_scatter_body(eps, nsc):
    def _scatter_body(m_hbm, dstr_hbm, init_hbm, agg_hbm, idx_v, buf_v, buf2_v,
                      agg_sh, ls0, ls1, ss0, ss1):
        cid = lax.axis_index("c")
        sid = lax.axis_index("s")
        col0 = cid * HHALF
        pltpu.sync_copy(init_hbm.at[pl.ds(sid * RSUB, RSUB), pl.ds(col0, HHALF)],
                        agg_sh.at[pl.ds(sid * RSUB, RSUB)])

        @pl.when(sid == 15)
        def _():
            pltpu.sync_copy(init_hbm.at[pl.ds(16 * RSUB, RTAIL), pl.ds(col0, HHALF)],
                            agg_sh.at[pl.ds(16 * RSUB, RTAIL)])

        plsc.subcore_barrier()
        pltpu.sync_copy(dstr_hbm.at[sid], idx_v)
        ebase = sid * eps

        buf = (buf_v, buf2_v)
        lsem = (ls0, ls1)
        ssem = (ss0, ss1)

        def l_desc(g, slot):
            return pltpu.make_async_copy(
                m_hbm.at[pl.ds(ebase + g * SCH, SCH), pl.ds(col0, HHALF)],
                buf[slot], lsem[slot])

        def s_desc(g, slot):
            return pltpu.make_async_copy(buf[slot], agg_sh.at[idx_v.at[g]],
                                         ssem[slot])

        l_desc(0, 0).start()

        def phase(g, slot):
            @pl.when(g + 1 < nsc)
            def _():
                @pl.when(g >= 1)
                def _():
                    s_desc(g - 1, 1 - slot).wait()
                l_desc(g + 1, 1 - slot).start()

            l_desc(g, slot).wait()
            s_desc(g, slot).start(add=True)

        @pl.loop(0, nsc // 2)
        def _(i):
            phase(2 * i, 0)
            phase(2 * i + 1, 1)

        if nsc % 2:
            phase(nsc - 1, (nsc - 1) % 2)

        for k in (nsc - 2, nsc - 1):
            s_desc(k, k % 2).wait()

        plsc.subcore_barrier()
        pltpu.sync_copy(agg_sh.at[pl.ds(sid * RSUB, RSUB)],
                        agg_hbm.at[pl.ds(sid * RSUB, RSUB), pl.ds(col0, HHALF)])

        @pl.when(sid == 15)
        def _():
            pltpu.sync_copy(agg_sh.at[pl.ds(16 * RSUB, RTAIL)],
                            agg_hbm.at[pl.ds(16 * RSUB, RTAIL), pl.ds(col0, HHALF)])

    return _scatter_body


def _scalar_body(src_hbm, dst_hbm, s_hbm, p_hbm, part_hbm,
                 src_v, dst_v, s_v, p_v, deg_v, cnt_v, ss_v, sp_v, ssp_v):
    wid = lax.axis_index("s") * 2 + lax.axis_index("c")
    base = wid * EPW
    pltpu.sync_copy(src_hbm.at[pl.ds(base, EPW)], src_v)
    pltpu.sync_copy(dst_hbm.at[pl.ds(base, EPW)], dst_v)
    pltpu.sync_copy(s_hbm.at[pl.ds(base, EPW)], s_v)
    pltpu.sync_copy(p_hbm.at[pl.ds(base, EPW)], p_v)

    zero16 = jnp.zeros((16,), jnp.float32)

    for ref in (deg_v, cnt_v, ss_v, sp_v, ssp_v):
        @pl.loop(0, N // 16)
        def _(i, ref=ref):
            ref[pl.ds(i * 16, 16)] = zero16

    ones = jnp.ones((16,), jnp.float32)

    @pl.loop(0, EPW // 16)
    def _(i):
        off = i * 16
        si = src_v[pl.ds(off, 16)]
        di = dst_v[pl.ds(off, 16)]
        sv = s_v[pl.ds(off, 16)]
        pv = p_v[pl.ds(off, 16)]
        plsc.addupdate_scatter(deg_v, [di], ones)
        plsc.addupdate_scatter(cnt_v, [si], ones)
        plsc.addupdate_scatter(ss_v, [si], sv)
        plsc.addupdate_scatter(sp_v, [si], pv)
        plsc.addupdate_scatter(ssp_v, [si], sv * pv)

    for k, ref in enumerate((deg_v, cnt_v, ss_v, sp_v, ssp_v)):
        pltpu.sync_copy(ref, part_hbm.at[pl.ds(wid * 5 * N + k * N, N)])


def _final_body(h_ref, agg_ref, part_ref, s_ref, p_ref, wc_ref, bc_ref, out_ref):
    parts = jnp.sum(part_ref[...], axis=0)
    deg = jnp.maximum(parts[0], 1.0)
    cnt_raw = parts[1]
    cnt = jnp.maximum(cnt_raw, 1.0)
    ssum = parts[2]
    psum = parts[3]
    spsum = parts[4]

    s = s_ref[...]
    p = p_ref[...]
    smin = jnp.min(s)
    smax = jnp.max(s)
    pmin = jnp.min(p)
    pmax = jnp.max(p)
    rs = smax - smin + 1e-12
    rp = pmax - pmin + 1e-12
    numer = spsum - pmin * ssum - smin * psum + cnt_raw * (smin * pmin)
    node_w = numer / (rs * rp) / cnt

    h2 = h_ref[...] + agg_ref[...] / deg[:, None]
    hw = h2 * (1.0 + node_w)[:, None]

    acc = hw[0:NC]
    for k in range(1, N // NC):
        acc = acc + hw[k * NC:(k + 1) * NC]
    tail = N - (N // NC) * NC
    acc = acc + jnp.concatenate(
        [hw[(N // NC) * NC:N], jnp.zeros((NC - tail, H), jnp.float32)], axis=0)

    cidx = lax.broadcasted_iota(jnp.int32, (NC, 1), 0)
    ccnt = jnp.where(cidx < tail, float(N // NC + 1), float(N // NC))
    sfeat = acc / ccnt
    z = jnp.dot(sfeat, wc_ref[...], preferred_element_type=jnp.float32) + bc_ref[...]
    out_ref[...] = jax.nn.sigmoid(jax.nn.sigmoid(z))


_SC_CALLS = None


def _make_gather_call(mesh, eh):
    eps = eh // 16
    ngf = eps // GCH
    gtail = eps - ngf * GCH
    return pl.kernel(
        _make_gather_body(eps, ngf, gtail),
        out_type=(jax.ShapeDtypeStruct((eh, H), jnp.float32),
                  jax.ShapeDtypeStruct((eh, H), jnp.float32)),
        mesh=mesh,
        scratch_types=[
            pltpu.VMEM((eps,), jnp.int32),
            pltpu.VMEM((eps,), jnp.int32),
        ] + [pltpu.VMEM((GCH, HHALF), jnp.float32)] * 3
          + [pltpu.VMEM_SHARED((N, HHALF), jnp.float32)]
          + [pltpu.SemaphoreType.DMA] * 6,
    )


def _make_scatter_call(mesh, eh):
    eps = eh // 16
    nsc = eps // SCH
    return pl.kernel(
        _make_scatter_body(eps, nsc),
        out_type=jax.ShapeDtypeStruct((N, H), jnp.float32),
        mesh=mesh,
        scratch_types=[
            pltpu.VMEM((nsc, SCH), jnp.int32),
            pltpu.VMEM((SCH, HHALF), jnp.float32),
            pltpu.VMEM((SCH, HHALF), jnp.float32),
            pltpu.VMEM_SHARED((N, HHALF), jnp.float32),
        ] + [pltpu.SemaphoreType.DMA] * 4,
    )


def _get_sc_calls():
    global _SC_CALLS
    if _SC_CALLS is not None:
        return _SC_CALLS
    mesh = plsc.VectorSubcoreMesh(core_axis_name="c", subcore_axis_name="s")

    gather1 = _make_gather_call(mesh, EH1)
    gather2 = _make_gather_call(mesh, EH2)
    scatter1 = _make_scatter_call(mesh, EH1)
    scatter2 = _make_scatter_call(mesh, EH2)

    scalar_call = pl.kernel(
        _scalar_body,
        out_type=jax.ShapeDtypeStruct((NWORK * 5 * N,), jnp.float32),
        mesh=mesh,
        compiler_params=pltpu.CompilerParams(needs_layout_passes=False),
        scratch_types=[
            pltpu.VMEM((EPW,), jnp.int32),
            pltpu.VMEM((EPW,), jnp.int32),
            pltpu.VMEM((EPW,), jnp.float32),
            pltpu.VMEM((EPW,), jnp.float32),
            pltpu.VMEM((N,), jnp.float32),
            pltpu.VMEM((N,), jnp.float32),
            pltpu.VMEM((N,), jnp.float32),
            pltpu.VMEM((N,), jnp.float32),
            pltpu.VMEM((N,), jnp.float32),
        ],
    )
    _SC_CALLS = (gather1, gather2, scatter1, scatter2, scalar_call)
    return _SC_CALLS

NB = 1000
EB = 3200


def _full2(i):
    return (0, 0)


def _edge_pass(eh, edge_attr, Asrc, Bdst, We, be, WmC, bm, Wsp, bsp):
    return pl.pallas_call(
        _edge_body,
        grid=(eh // EB,),
        in_specs=[
            pl.BlockSpec((EB, DE), lambda i: (i, 0)),
            pl.BlockSpec((EB, H), lambda i: (i, 0)),
            pl.BlockSpec((EB, H), lambda i: (i, 0)),
            pl.BlockSpec((DE, H), _full2), pl.BlockSpec((1, H), _full2),
            pl.BlockSpec((H, H), _full2), pl.BlockSpec((1, H), _full2),
            pl.BlockSpec((H, 2), _full2), pl.BlockSpec((1, 2), _full2),
        ],
        out_specs=[
            pl.BlockSpec((EB, H), lambda i: (i, 0)),
            pl.BlockSpec((1, 1, EB), lambda i: (i, 0, 0)),
            pl.BlockSpec((1, 1, EB), lambda i: (i, 0, 0)),
        ],
        out_shape=[
            jax.ShapeDtypeStruct((eh, H), jnp.float32),
            jax.ShapeDtypeStruct((eh // EB, 1, EB), jnp.float32),
            jax.ShapeDtypeStruct((eh // EB, 1, EB), jnp.float32),
        ],
    )(edge_attr, Asrc, Bdst, We, be, WmC, bm, Wsp, bsp)


@jax.jit
def _run(x, src, dst, edge_attr, Wn1, bn1, Wn2, bn2, We, be,
         WmA, WmB, WmC, bm, Wsp, bsp, Wc, bc, zeros_full):
    gather1, gather2, scatter1, scatter2, scalar_call = _get_sc_calls()
    h, A, B = pl.pallas_call(
        _node_prep_body,
        grid=(N // NB,),
        in_specs=[
            pl.BlockSpec((NB, D), lambda i: (i, 0)),
            pl.BlockSpec((D, H), _full2), pl.BlockSpec((1, H), _full2),
            pl.BlockSpec((H, H), _full2), pl.BlockSpec((1, H), _full2),
            pl.BlockSpec((H, H), _full2), pl.BlockSpec((H, H), _full2),
        ],
        out_specs=[pl.BlockSpec((NB, H), lambda i: (i, 0))] * 3,
        out_shape=[jax.ShapeDtypeStruct((N, H), jnp.float32)] * 3,
    )(x, Wn1, bn1, Wn2, bn2, WmA, WmB)

    src1, src2 = src[:EH1], src[EH1:]
    dst1, dst2 = dst[:EH1], dst[EH1:]
    dstr1 = dst1.reshape(16, (EH1 // 16) // SCH, SCH)
    dstr2 = dst2.reshape(16, (EH2 // 16) // SCH, SCH)

    Asrc1, Bdst1 = gather1(A, B, src1, dst1)
    Asrc2, Bdst2 = gather2(A, B, src2, dst2)

    ea1, ea2 = edge_attr[:EH1], edge_attr[EH1:]
    m1, s1, p1 = _edge_pass(EH1, ea1, Asrc1, Bdst1, We, be, WmC, bm, Wsp, bsp)
    agg1 = scatter1(m1, dstr1, zeros_full)
    m2, s2, p2 = _edge_pass(EH2, ea2, Asrc2, Bdst2, We, be, WmC, bm, Wsp, bsp)
    agg = scatter2(m2, dstr2, agg1)

    s_e = jnp.concatenate([s1.reshape(EH1), s2.reshape(EH2)])
    p_e = jnp.concatenate([p1.reshape(EH1), p2.reshape(EH2)])
    parts = scalar_call(src, dst, s_e, p_e).reshape(NWORK, 5, N)

    out = pl.pallas_call(
        _final_body,
        in_specs=[
            pl.BlockSpec((N, H)),
            pl.BlockSpec((N, H)),
            pl.BlockSpec((NWORK, 5, N)),
            pl.BlockSpec((E,)),
            pl.BlockSpec((E,)),
            pl.BlockSpec((H, 1)),
            pl.BlockSpec((1, 1)),
        ],
        out_specs=pl.BlockSpec((NC, 1)),
        out_shape=jax.ShapeDtypeStruct((NC, 1), jnp.float32),
    )(h, agg, parts, s_e, p_e, Wc, bc)
    return out


def kernel(x, edge_index, edge_attr, Wn1, bn1, Wn2, bn2, We, be, Wm, bm,
           Ws, bs, Wp, bp, Wc, bc):
    src = edge_index[0].astype(jnp.int32)
    dst = edge_index[1].astype(jnp.int32)
    WmA = Wm[:H]
    WmB = Wm[H:2 * H]
    WmC = Wm[2 * H:]
    Wsp = jnp.concatenate([Ws, Wp], axis=1)
    bsp = jnp.concatenate([bs, bp]).reshape(1, 2)
    zeros_full = jnp.zeros((N, H), jnp.float32)
    return _run(x, src, dst, edge_attr,
                Wn1, bn1.reshape(1, H), Wn2, bn2.reshape(1, H),
                We, be.reshape(1, H), WmA, WmB, WmC, bm.reshape(1, H),
                Wsp, bsp, Wc, bc.reshape(1, 1), zeros_full)
